# baseline (speedup 1.0000x reference)
import functools

import jax
import jax.numpy as jnp
from jax.experimental import pallas as pl
from jax.experimental.pallas import tpu as pltpu

NR, NA = 1024, 8192
FR = 192
ID_DIM = 64
HID = 256
HEADS = 4
HEAD_DIM = HID // HEADS
NGRAPH = 16
TOPK = 10
KPAD = 16


def _dot(a, b):
    return jax.lax.dot_general(a, b, (((1,), (0,)), ((), ())),
                               preferred_element_type=jnp.float32)


def _proj_kernel(xe_ref, xf_ref, wq_ref, bq_ref, wv_ref, bv_ref,
                 ws_ref, bs_ref, qo_ref, vo_ref, so_ref):
    h = pl.program_id(1)
    xe = xe_ref[...]
    xf = xf_ref[...]
    wq = wq_ref[0]
    qo_ref[0] = _dot(xe, wq[0:ID_DIM]) + _dot(xf, wq[ID_DIM:]) + bq_ref[0]
    wv = wv_ref[0]
    vo_ref[0] = _dot(xe, wv[0:ID_DIM]) + _dot(xf, wv[ID_DIM:]) + bv_ref[0]

    @pl.when(h == 0)
    def _():
        ws = ws_ref[...]
        so_ref[...] = _dot(xe, ws[0:ID_DIM]) + _dot(xf, ws[ID_DIM:]) + bs_ref[...]


def _proj_side(emb, feat, wq, bq, wv, bv, wself, bself, bm):
    n, cin = emb.shape[0], ID_DIM + FR
    wq3 = wq.T.reshape(cin, HEADS, HEAD_DIM).transpose(1, 0, 2)
    wv3 = wv.T.reshape(cin, HEADS, HEAD_DIM).transpose(1, 0, 2)
    bq3 = bq.reshape(HEADS, 1, HEAD_DIM)
    bv3 = bv.reshape(HEADS, 1, HEAD_DIM)
    nb = n // bm
    return pl.pallas_call(
        _proj_kernel,
        grid=(nb, HEADS),
        in_specs=[
            pl.BlockSpec((bm, ID_DIM), lambda b, h: (b, 0)),
            pl.BlockSpec((bm, FR), lambda b, h: (b, 0)),
            pl.BlockSpec((1, cin, HEAD_DIM), lambda b, h: (h, 0, 0)),
            pl.BlockSpec((1, 1, HEAD_DIM), lambda b, h: (h, 0, 0)),
            pl.BlockSpec((1, cin, HEAD_DIM), lambda b, h: (h, 0, 0)),
            pl.BlockSpec((1, 1, HEAD_DIM), lambda b, h: (h, 0, 0)),
            pl.BlockSpec((cin, HID), lambda b, h: (0, 0)),
            pl.BlockSpec((1, HID), lambda b, h: (0, 0)),
        ],
        out_specs=[
            pl.BlockSpec((1, bm, HEAD_DIM), lambda b, h: (h, b, 0)),
            pl.BlockSpec((1, bm, HEAD_DIM), lambda b, h: (h, b, 0)),
            pl.BlockSpec((bm, HID), lambda b, h: (b, 0)),
        ],
        out_shape=[
            jax.ShapeDtypeStruct((HEADS, n, HEAD_DIM), jnp.float32),
            jax.ShapeDtypeStruct((HEADS, n, HEAD_DIM), jnp.float32),
            jax.ShapeDtypeStruct((n, HID), jnp.float32),
        ],
    )(emb, feat, wq3, bq3, wv3, bv3, wself.T, bself.reshape(1, HID))


def _attn_body(t, q_ref, k_ref, m_ref, av_ref, rv_ref, r2a_ref, a2r_ref,
               sw_ref, sr_ref, bn):
    s = jax.lax.dot_general(
        q_ref[0], k_ref[0], (((1,), (1,)), ((), ())),
        preferred_element_type=jnp.float32)
    sw_ref[...] = s * m_ref[...]

    lane16 = jax.lax.broadcasted_iota(jnp.int32, (bn, KPAD), 1)
    s0 = sr_ref[...]
    mx = jnp.max(s0, axis=1, keepdims=True)
    vals = jnp.where(lane16 == 0, mx, -jnp.inf)
    for j in range(1, TOPK):
        mx = jnp.max(jnp.where(s0 < mx, s0, -jnp.inf), axis=1, keepdims=True)
        vals = jnp.where(lane16 == j, mx, vals)
    v10 = mx
    sg = jax.nn.sigmoid(vals)
    smax = sg[:, 0:1]
    e = jnp.where(lane16 < TOPK, jnp.exp(sg - smax), 0.0)
    rz = 1.0 / jnp.sum(e, axis=1, keepdims=True)
    attnw = jnp.where(
        (s0 >= v10) & (s0 > 0.0),
        jnp.exp(jax.nn.sigmoid(s0) - smax) * rz, 0.0)
    r2a_ref[0] = jax.lax.dot_general(
        attnw, av_ref[0], (((1,), (0,)), ((), ())),
        preferred_element_type=jnp.float32)

    @pl.when(t > 0)
    def _():
        hp = jnp.maximum(t - 1, 0) % HEADS
        a2r_ref[hp] += jax.lax.dot_general(
            attnw, rv_ref[0], (((0,), (0,)), ((), ())),
            preferred_element_type=jnp.float32)


def _attn_kernel(q_ref, k_ref, m_ref, av_ref, rv_ref, r2a_ref, a2r_ref,
                 s0_ref, s1_ref, *, bn, nsteps):
    t = pl.program_id(0)

    @pl.when(t == 0)
    def _():
        a2r_ref[...] = jnp.zeros_like(a2r_ref)

    @pl.when(t % 2 == 0)
    def _():
        _attn_body(t, q_ref, k_ref, m_ref, av_ref, rv_ref, r2a_ref, a2r_ref,
                   s0_ref, s1_ref, bn)

    @pl.when(t % 2 == 1)
    def _():
        _attn_body(t, q_ref, k_ref, m_ref, av_ref, rv_ref, r2a_ref, a2r_ref,
                   s1_ref, s0_ref, bn)


def _attn(q3, k3, av3, rv3, mask2d, bn=128):
    nb = NR // bn
    nbh = nb * HEADS

    def _tc(t):
        return jnp.minimum(t, nbh - 1)

    def _te(t):
        return jnp.maximum(t - 1, 0)

    return pl.pallas_call(
        functools.partial(_attn_kernel, bn=bn, nsteps=nbh + 1),
        grid=(nbh + 1,),
        in_specs=[
            pl.BlockSpec((1, bn, HEAD_DIM),
                         lambda t: (_tc(t) % HEADS, _tc(t) // HEADS, 0)),
            pl.BlockSpec((1, NA, HEAD_DIM), lambda t: (_tc(t) % HEADS, 0, 0)),
            pl.BlockSpec((bn, NA), lambda t: (_tc(t) // HEADS, 0)),
            pl.BlockSpec((1, NA, HEAD_DIM), lambda t: (_te(t) % HEADS, 0, 0)),
            pl.BlockSpec((1, bn, HEAD_DIM),
                         lambda t: (_te(t) % HEADS, _te(t) // HEADS, 0)),
        ],
        out_specs=[
            pl.BlockSpec((1, bn, HEAD_DIM),
                         lambda t: (_te(t) % HEADS, _te(t) // HEADS, 0)),
            pl.BlockSpec((HEADS, NA, HEAD_DIM), lambda t: (0, 0, 0)),
        ],
        out_shape=[
            jax.ShapeDtypeStruct((HEADS, NR, HEAD_DIM), jnp.float32),
            jax.ShapeDtypeStruct((HEADS, NA, HEAD_DIM), jnp.float32),
        ],
        scratch_shapes=[pltpu.VMEM((bn, NA), jnp.float32),
                        pltpu.VMEM((bn, NA), jnp.float32)],
    )(q3, k3, mask2d, av3, rv3)


def _reduce_kernel(x3_ref, sf_ref, ids_ref, wo_ref, bo_ref,
                   rw1_ref, rw2_ref, rb_ref, sum_ref, cnt_ref):
    m = pl.program_id(0)

    @pl.when(m == 0)
    def _():
        sum_ref[...] = jnp.zeros_like(sum_ref)
        cnt_ref[...] = jnp.zeros_like(cnt_ref)

    tmp = bo_ref[...]
    for h in range(HEADS):
        tmp = tmp + _dot(x3_ref[h], wo_ref[h])
    red = _dot(tmp, rw1_ref[...]) + _dot(sf_ref[...], rw2_ref[...]) + rb_ref[...]
    seg = jax.lax.broadcasted_iota(jnp.int32, (NGRAPH, 1), 0)
    oh = (ids_ref[...] == seg).astype(jnp.float32)
    sum_ref[...] += jax.lax.dot_general(
        oh, red, (((1,), (0,)), ((), ())), preferred_element_type=jnp.float32)
    cnt_ref[...] += jnp.broadcast_to(
        jnp.sum(oh, axis=1, keepdims=True), cnt_ref.shape)


def _reduce_side(x3, selfp, ids, wo, bo, rw, rb, bm):
    n = selfp.shape[0]
    nb = n // bm
    wo3 = wo.T.reshape(HEADS, HEAD_DIM, HID)
    rwt = rw.T
    return pl.pallas_call(
        _reduce_kernel,
        grid=(nb,),
        in_specs=[
            pl.BlockSpec((HEADS, bm, HEAD_DIM), lambda m: (0, m, 0)),
            pl.BlockSpec((bm, HID), lambda m: (m, 0)),
            pl.BlockSpec((1, bm), lambda m: (0, m)),
            pl.BlockSpec((HEADS, HEAD_DIM, HID), lambda m: (0, 0, 0)),
            pl.BlockSpec((1, HID), lambda m: (0, 0)),
            pl.BlockSpec((HID, HID), lambda m: (0, 0)),
            pl.BlockSpec((HID, HID), lambda m: (0, 0)),
            pl.BlockSpec((1, HID), lambda m: (0, 0)),
        ],
        out_specs=[
            pl.BlockSpec((NGRAPH, HID), lambda m: (0, 0)),
            pl.BlockSpec((NGRAPH, 128), lambda m: (0, 0)),
        ],
        out_shape=[
            jax.ShapeDtypeStruct((NGRAPH, HID), jnp.float32),
            jax.ShapeDtypeStruct((NGRAPH, 128), jnp.float32),
        ],
    )(x3, selfp, ids.reshape(1, n).astype(jnp.int32), wo3,
      bo.reshape(1, HID), rwt[:HID], rwt[HID:], rb.reshape(1, HID))


def _mlp_kernel(sr_ref, cr_ref, sa_ref, ca_ref,
                w1_ref, b1_ref, w2_ref, b2_ref, w3_ref, b3_ref, o_ref):
    mean_r = sr_ref[...] / jnp.maximum(cr_ref[:, 0:1], 1.0)
    mean_a = sa_ref[...] / jnp.maximum(ca_ref[:, 0:1], 1.0)
    x = jnp.concatenate([mean_r, mean_a], axis=1)
    x = jnp.maximum(_dot(x, w1_ref[...]) + b1_ref[...], 0.0)
    x = jnp.maximum(_dot(x, w2_ref[...]) + b2_ref[...], 0.0)
    o_ref[...] = _dot(x, w3_ref[...]) + b3_ref[...]


def _head_mlp(sum_r, cnt_r, sum_a, cnt_a, p):
    args = (sum_r, cnt_r, sum_a, cnt_a,
            p["fc1_w"].T, p["fc1_b"].reshape(1, -1),
            p["fc2_w"].T, p["fc2_b"].reshape(1, -1),
            p["fc3_w"].T, p["fc3_b"].reshape(1, -1))
    return pl.pallas_call(
        _mlp_kernel,
        out_shape=jax.ShapeDtypeStruct((NGRAPH, 2), jnp.float32),
    )(*args)


def kernel(rna_ids, rna_feat, atac_ids, atac_feat, chrom_mask,
           rna_batch, atac_batch, params):
    p = params
    emb_r = p["rna_emb"][rna_ids]
    emb_a = p["atac_emb"][atac_ids]
    mask2d = chrom_mask[..., 0]

    q3, rv3, rself = _proj_side(
        emb_r, rna_feat, p["rna_query_w"], p["rna_query_b"],
        p["rna_value_w"], p["rna_value_b"],
        p["rna_self_w"], p["rna_self_b"], bm=1024)
    k3, av3, aself = _proj_side(
        emb_a, atac_feat, p["atac_key_w"], p["atac_key_b"],
        p["atac_value_w"], p["atac_value_b"],
        p["atac_self_w"], p["atac_self_b"], bm=1024)

    r2a3, a2r3 = _attn(q3, k3, av3, rv3, mask2d)

    sum_r, cnt_r = _reduce_side(r2a3, rself, rna_batch,
                                p["rna_out_w"], p["rna_out_b"],
                                p["red_rna_w"], p["red_rna_b"], bm=1024)
    sum_a, cnt_a = _reduce_side(a2r3, aself, atac_batch,
                                p["atac_out_w"], p["atac_out_b"],
                                p["red_atac_w"], p["red_atac_b"], bm=1024)
    return _head_mlp(sum_r, cnt_r, sum_a, cnt_a, p)


# speedup vs baseline: 1.0806x; 1.0806x over previous
import functools

import jax
import jax.numpy as jnp
from jax.experimental import pallas as pl
from jax.experimental.pallas import tpu as pltpu

NR, NA = 1024, 8192
FR = 192
ID_DIM = 64
HID = 256
HEADS = 4
HEAD_DIM = HID // HEADS
NGRAPH = 16
TOPK = 10
KPAD = 16


def _dot(a, b):
    return jax.lax.dot_general(a, b, (((1,), (0,)), ((), ())),
                               preferred_element_type=jnp.float32)


def _proj_kernel(xe_ref, xf_ref, we_ref, wf_ref, b_ref, qo_ref, vo_ref, so_ref):
    y = (_dot(xe_ref[...], we_ref[...]) + _dot(xf_ref[...], wf_ref[...])
         + b_ref[...])
    for h in range(HEADS):
        qo_ref[h] = y[:, h * HEAD_DIM:(h + 1) * HEAD_DIM]
        vo_ref[h] = y[:, HID + h * HEAD_DIM:HID + (h + 1) * HEAD_DIM]
    so_ref[...] = y[:, 2 * HID:]


def _proj_side(emb, feat, wq, bq, wv, bv, wself, bself, bm):
    n, cin = emb.shape[0], ID_DIM + FR
    wcat = jnp.concatenate([wq, wv, wself], axis=0).T
    bcat = jnp.concatenate([bq, bv, bself], axis=0).reshape(1, 3 * HID)
    nb = n // bm
    return pl.pallas_call(
        _proj_kernel,
        grid=(nb,),
        in_specs=[
            pl.BlockSpec((bm, ID_DIM), lambda b: (b, 0)),
            pl.BlockSpec((bm, FR), lambda b: (b, 0)),
            pl.BlockSpec((ID_DIM, 3 * HID), lambda b: (0, 0)),
            pl.BlockSpec((FR, 3 * HID), lambda b: (0, 0)),
            pl.BlockSpec((1, 3 * HID), lambda b: (0, 0)),
        ],
        out_specs=[
            pl.BlockSpec((HEADS, bm, HEAD_DIM), lambda b: (0, b, 0)),
            pl.BlockSpec((HEADS, bm, HEAD_DIM), lambda b: (0, b, 0)),
            pl.BlockSpec((bm, HID), lambda b: (b, 0)),
        ],
        out_shape=[
            jax.ShapeDtypeStruct((HEADS, n, HEAD_DIM), jnp.float32),
            jax.ShapeDtypeStruct((HEADS, n, HEAD_DIM), jnp.float32),
            jax.ShapeDtypeStruct((n, HID), jnp.float32),
        ],
    )(emb, feat, wcat[:ID_DIM], wcat[ID_DIM:], bcat)


def _attn_kernel(q_ref, k_ref, m_ref, av_ref, rv_ref, r2a_ref, a2r_ref,
                 s0_ref, *, bn):
    b = pl.program_id(0)
    h = pl.program_id(1)

    @pl.when((b == 0) & (h == 0))
    def _():
        a2r_ref[...] = jnp.zeros_like(a2r_ref)

    s = jax.lax.dot_general(
        q_ref[0], k_ref[0], (((1,), (1,)), ((), ())),
        preferred_element_type=jnp.float32)
    s0_ref[...] = s * m_ref[...]
    lane16 = jax.lax.broadcasted_iota(jnp.int32, (bn, KPAD), 1)
    s0 = s0_ref[...]
    mx = jnp.max(s0, axis=1, keepdims=True)
    vals = jnp.where(lane16 == 0, mx, -jnp.inf)
    for j in range(1, TOPK):
        mx = jnp.max(jnp.where(s0 < mx, s0, -jnp.inf), axis=1, keepdims=True)
        vals = jnp.where(lane16 == j, mx, vals)
    v10 = mx
    sg = jax.nn.sigmoid(vals)
    smax = sg[:, 0:1]
    e = jnp.where(lane16 < TOPK, jnp.exp(sg - smax), 0.0)
    rz = 1.0 / jnp.sum(e, axis=1, keepdims=True)
    attnw = jnp.where(
        (s0 >= v10) & (s0 > 0.0),
        jnp.exp(jax.nn.sigmoid(s0) - smax) * rz, 0.0)
    r2a_ref[0] = jax.lax.dot_general(
        attnw, av_ref[0], (((1,), (0,)), ((), ())),
        preferred_element_type=jnp.float32)
    a2r_ref[h] += jax.lax.dot_general(
        attnw, rv_ref[0], (((0,), (0,)), ((), ())),
        preferred_element_type=jnp.float32)


def _attn(q3, k3, av3, rv3, mask2d, bn=128):
    nb = NR // bn
    return pl.pallas_call(
        functools.partial(_attn_kernel, bn=bn),
        grid=(nb, HEADS),
        in_specs=[
            pl.BlockSpec((1, bn, HEAD_DIM), lambda b, h: (h, b, 0)),
            pl.BlockSpec((1, NA, HEAD_DIM), lambda b, h: (h, 0, 0)),
            pl.BlockSpec((bn, NA), lambda b, h: (b, 0)),
            pl.BlockSpec((1, NA, HEAD_DIM), lambda b, h: (h, 0, 0)),
            pl.BlockSpec((1, bn, HEAD_DIM), lambda b, h: (h, b, 0)),
        ],
        out_specs=[
            pl.BlockSpec((1, bn, HEAD_DIM), lambda b, h: (h, b, 0)),
            pl.BlockSpec((HEADS, NA, HEAD_DIM), lambda b, h: (0, 0, 0)),
        ],
        out_shape=[
            jax.ShapeDtypeStruct((HEADS, NR, HEAD_DIM), jnp.float32),
            jax.ShapeDtypeStruct((HEADS, NA, HEAD_DIM), jnp.float32),
        ],
        scratch_shapes=[pltpu.VMEM((bn, NA), jnp.float32)],
    )(q3, k3, mask2d, av3, rv3)


def _reduce_kernel(x3_ref, sf_ref, ids_ref, wo_ref, bo_ref,
                   rw1_ref, rw2_ref, rb_ref, sum_ref, cnt_ref):
    m = pl.program_id(0)

    @pl.when(m == 0)
    def _():
        sum_ref[...] = jnp.zeros_like(sum_ref)
        cnt_ref[...] = jnp.zeros_like(cnt_ref)

    tmp = bo_ref[...]
    for h in range(HEADS):
        tmp = tmp + _dot(x3_ref[h], wo_ref[h])
    red = _dot(tmp, rw1_ref[...]) + _dot(sf_ref[...], rw2_ref[...]) + rb_ref[...]
    seg = jax.lax.broadcasted_iota(jnp.int32, (NGRAPH, 1), 0)
    oh = (ids_ref[...] == seg).astype(jnp.float32)
    sum_ref[...] += jax.lax.dot_general(
        oh, red, (((1,), (0,)), ((), ())), preferred_element_type=jnp.float32)
    cnt_ref[...] += jnp.broadcast_to(
        jnp.sum(oh, axis=1, keepdims=True), cnt_ref.shape)


def _reduce_side(x3, selfp, ids, wo, bo, rw, rb, bm):
    n = selfp.shape[0]
    nb = n // bm
    wo3 = wo.T.reshape(HEADS, HEAD_DIM, HID)
    rwt = rw.T
    return pl.pallas_call(
        _reduce_kernel,
        grid=(nb,),
        in_specs=[
            pl.BlockSpec((HEADS, bm, HEAD_DIM), lambda m: (0, m, 0)),
            pl.BlockSpec((bm, HID), lambda m: (m, 0)),
            pl.BlockSpec((1, bm), lambda m: (0, m)),
            pl.BlockSpec((HEADS, HEAD_DIM, HID), lambda m: (0, 0, 0)),
            pl.BlockSpec((1, HID), lambda m: (0, 0)),
            pl.BlockSpec((HID, HID), lambda m: (0, 0)),
            pl.BlockSpec((HID, HID), lambda m: (0, 0)),
            pl.BlockSpec((1, HID), lambda m: (0, 0)),
        ],
        out_specs=[
            pl.BlockSpec((NGRAPH, HID), lambda m: (0, 0)),
            pl.BlockSpec((NGRAPH, 128), lambda m: (0, 0)),
        ],
        out_shape=[
            jax.ShapeDtypeStruct((NGRAPH, HID), jnp.float32),
            jax.ShapeDtypeStruct((NGRAPH, 128), jnp.float32),
        ],
    )(x3, selfp, ids.reshape(1, n).astype(jnp.int32), wo3,
      bo.reshape(1, HID), rwt[:HID], rwt[HID:], rb.reshape(1, HID))


def _mlp_kernel(sr_ref, cr_ref, sa_ref, ca_ref,
                w1_ref, b1_ref, w2_ref, b2_ref, w3_ref, b3_ref, o_ref):
    mean_r = sr_ref[...] / jnp.maximum(cr_ref[:, 0:1], 1.0)
    mean_a = sa_ref[...] / jnp.maximum(ca_ref[:, 0:1], 1.0)
    x = jnp.concatenate([mean_r, mean_a], axis=1)
    x = jnp.maximum(_dot(x, w1_ref[...]) + b1_ref[...], 0.0)
    x = jnp.maximum(_dot(x, w2_ref[...]) + b2_ref[...], 0.0)
    o_ref[...] = _dot(x, w3_ref[...]) + b3_ref[...]


def _head_mlp(sum_r, cnt_r, sum_a, cnt_a, p):
    args = (sum_r, cnt_r, sum_a, cnt_a,
            p["fc1_w"].T, p["fc1_b"].reshape(1, -1),
            p["fc2_w"].T, p["fc2_b"].reshape(1, -1),
            p["fc3_w"].T, p["fc3_b"].reshape(1, -1))
    return pl.pallas_call(
        _mlp_kernel,
        out_shape=jax.ShapeDtypeStruct((NGRAPH, 2), jnp.float32),
    )(*args)


def kernel(rna_ids, rna_feat, atac_ids, atac_feat, chrom_mask,
           rna_batch, atac_batch, params):
    p = params
    emb_r = p["rna_emb"][rna_ids]
    emb_a = p["atac_emb"][atac_ids]
    mask2d = chrom_mask[..., 0]

    q3, rv3, rself = _proj_side(
        emb_r, rna_feat, p["rna_query_w"], p["rna_query_b"],
        p["rna_value_w"], p["rna_value_b"],
        p["rna_self_w"], p["rna_self_b"], bm=1024)
    k3, av3, aself = _proj_side(
        emb_a, atac_feat, p["atac_key_w"], p["atac_key_b"],
        p["atac_value_w"], p["atac_value_b"],
        p["atac_self_w"], p["atac_self_b"], bm=1024)

    r2a3, a2r3 = _attn(q3, k3, av3, rv3, mask2d)

    sum_r, cnt_r = _reduce_side(r2a3, rself, rna_batch,
                                p["rna_out_w"], p["rna_out_b"],
                                p["red_rna_w"], p["red_rna_b"], bm=1024)
    sum_a, cnt_a = _reduce_side(a2r3, aself, atac_batch,
                                p["atac_out_w"], p["atac_out_b"],
                                p["red_atac_w"], p["red_atac_b"], bm=1024)
    return _head_mlp(sum_r, cnt_r, sum_a, cnt_a, p)
